# baseline (speedup 1.0000x reference)
import jax
import jax.numpy as jnp
from jax import lax
from jax.experimental import pallas as pl
from jax.experimental.pallas import tpu as pltpu
from jax.experimental.pallas import tpu_sc as plsc

N = 10000
E = 160000
D = 256
DE = 16

NC = 2
NS = 16
L = 16

NSLICE = 16
OWN = 5120
N_PAD = NC * OWN
DUMP = OWN
SP_ROWS = OWN + 8

C = 640
NCHUNK = E // C
GQ = C // 128

_HIGH = lax.Precision.HIGHEST


def _front_body(x_ref, wlt_ref, w1ht_ref, b1_ref, h_ref, g_ref):
    h = jnp.maximum(
        jnp.dot(x_ref[...], wlt_ref[...], preferred_element_type=jnp.float32,
                precision=_HIGH), 0.0)
    h_ref[...] = h
    g_ref[...] = jnp.dot(h, w1ht_ref[...], preferred_element_type=jnp.float32,
                         precision=_HIGH) + b1_ref[...]


def _edge_body(ea_ref, w1et_ref, out_ref):
    out_ref[...] = jnp.dot(ea_ref[...], w1et_ref[...],
                           preferred_element_type=jnp.float32, precision=_HIGH)


def _final_body(ns_ref, h_ref, w2nt_ref, w2ht_ref, b2_ref, o_ref):
    acc = jnp.dot(ns_ref[...], w2nt_ref[...],
                  preferred_element_type=jnp.float32, precision=_HIGH)
    acc = acc + jnp.dot(h_ref[...], w2ht_ref[...],
                        preferred_element_type=jnp.float32, precision=_HIGH)
    o_ref[...] = jnp.maximum(acc + b2_ref[...], 0.0)


def _sc_body(g2_hbm, ea3_hbm, src_hbm, dst_hbm, out_hbm,
             srcv, ldst, eabuf, gbuf, acc, sem):
    c = lax.axis_index("c")
    s = lax.axis_index("s")
    base = c * OWN

    zero = jnp.zeros((L,), jnp.float32)
    zero_i = jnp.zeros((L,), jnp.int32)
    iota = jax.lax.iota(jnp.int32, L)

    @pl.loop(0, SP_ROWS, unroll=8)
    def _zero(r):
        acc[r, pl.ds(0, L)] = zero

    @pl.loop(0, NCHUNK)
    def _chunk(j):
        pltpu.sync_copy(src_hbm.at[pl.ds(j, 1)], srcv)
        pltpu.sync_copy(dst_hbm.at[pl.ds(j, 1)], ldst)

        for a in range(C // L):
            sl = pl.ds(a * L, L)
            srcv[0, sl] = srcv[0, sl] + s * N
            d = ldst[0, sl] - base
            ok = (d >= 0) & (d < OWN)
            ldst[0, sl] = jnp.where(ok, d, DUMP)

        cps = [
            pltpu.async_copy(g2_hbm.at[srcv.at[0, pl.ds(q * 128, 128)]],
                             gbuf.at[pl.ds(q * 128, 128)], sem)
            for q in range(GQ)
        ]
        pltpu.sync_copy(ea3_hbm.at[pl.ds(j * C, C), s], eabuf)
        for cp in cps:
            cp.wait()

        @pl.loop(0, C, unroll=8)
        def _edge(r):
            rowv = plsc.load_gather(
                ldst, [zero_i, jnp.full((L,), r, jnp.int32)])
            m = jnp.maximum(gbuf[r, pl.ds(0, L)] + eabuf[r, pl.ds(0, L)], 0.0)
            plsc.addupdate_scatter(acc, [rowv, iota], m)

    pltpu.sync_copy(acc.at[pl.ds(0, OWN)], out_hbm.at[s, pl.ds(base, OWN)])


def _sc_aggregate(g2, ea3, src2, dst2):
    mesh = plsc.VectorSubcoreMesh(core_axis_name="c", subcore_axis_name="s",
                                  num_cores=NC, num_subcores=NS)
    run = pl.kernel(
        _sc_body,
        out_type=jax.ShapeDtypeStruct((NSLICE, N_PAD, L), jnp.float32),
        mesh=mesh,
        scratch_types=[
            pltpu.VMEM((1, C), jnp.int32),
            pltpu.VMEM((1, C), jnp.int32),
            pltpu.VMEM((C, L), jnp.float32),
            pltpu.VMEM((C, L), jnp.float32),
            pltpu.VMEM((SP_ROWS, L), jnp.float32),
            pltpu.SemaphoreType.DMA,
        ],
        compiler_params=pltpu.CompilerParams(needs_layout_passes=False,
                                             use_tc_tiling_on_sc=False),
    )
    return run(g2, ea3, src2, dst2)


def kernel(x, edge_index, edge_attr, W_lin, W1, b1, W2, b2):
    src2 = edge_index[0].astype(jnp.int32).reshape(NCHUNK, C)
    dst2 = edge_index[1].astype(jnp.int32).reshape(NCHUNK, C)

    wlt = W_lin.T
    w1ht = W1[:, :D].T
    w1et = W1[:, D:].T
    w2nt = W2[:, :D].T
    w2ht = W2[:, D:].T
    b1r = b1.reshape(1, D)
    b2r = b2.reshape(1, D)

    bn = 2000
    gn = N // bn
    full = pl.BlockSpec((None,) * 2, lambda i: (0, 0))

    h, g = pl.pallas_call(
        _front_body,
        grid=(gn,),
        in_specs=[
            pl.BlockSpec((bn, D), lambda i: (i, 0)),
            pl.BlockSpec((D, D), lambda i: (0, 0)),
            pl.BlockSpec((D, D), lambda i: (0, 0)),
            pl.BlockSpec((1, D), lambda i: (0, 0)),
        ],
        out_specs=[
            pl.BlockSpec((bn, D), lambda i: (i, 0)),
            pl.BlockSpec((bn, D), lambda i: (i, 0)),
        ],
        out_shape=[
            jax.ShapeDtypeStruct((N, D), jnp.float32),
            jax.ShapeDtypeStruct((N, D), jnp.float32),
        ],
    )(x, wlt, w1ht, b1r)

    be = 8000
    ea_proj = pl.pallas_call(
        _edge_body,
        grid=(E // be,),
        in_specs=[
            pl.BlockSpec((be, DE), lambda i: (i, 0)),
            pl.BlockSpec((DE, D), lambda i: (0, 0)),
        ],
        out_specs=pl.BlockSpec((be, D), lambda i: (i, 0)),
        out_shape=jax.ShapeDtypeStruct((E, D), jnp.float32),
    )(edge_attr, w1et)

    g2 = g.reshape(N, NSLICE, L).transpose(1, 0, 2).reshape(NSLICE * N, L)
    ea3 = ea_proj.reshape(E, NSLICE, L)
    out_blk = _sc_aggregate(g2, ea3, src2, dst2)
    ns = out_blk.transpose(1, 0, 2).reshape(N_PAD, D)[:N]

    out = pl.pallas_call(
        _final_body,
        grid=(gn,),
        in_specs=[
            pl.BlockSpec((bn, D), lambda i: (i, 0)),
            pl.BlockSpec((bn, D), lambda i: (i, 0)),
            pl.BlockSpec((D, D), lambda i: (0, 0)),
            pl.BlockSpec((D, D), lambda i: (0, 0)),
            pl.BlockSpec((1, D), lambda i: (0, 0)),
        ],
        out_specs=pl.BlockSpec((bn, D), lambda i: (i, 0)),
        out_shape=jax.ShapeDtypeStruct((N, D), jnp.float32),
    )(ns, h, w2nt, w2ht, b2r)

    return out


# speedup vs baseline: 1.1155x; 1.1155x over previous
import jax
import jax.numpy as jnp
from jax import lax
from jax.experimental import pallas as pl
from jax.experimental.pallas import tpu as pltpu
from jax.experimental.pallas import tpu_sc as plsc

N = 10000
E = 160000
D = 256
DE = 16

NC = 2
NS = 16
L = 16

NSLICE = 16
OWN = 5120
N_PAD = NC * OWN
DUMP = OWN
SP_ROWS = OWN + 8

C = 640
NCHUNK = E // C
GQ = C // 128

_HIGH = lax.Precision.HIGHEST


def _front_body(x_ref, wlt_ref, w1ht_ref, b1_ref, h_ref, g_ref):
    h = jnp.maximum(
        jnp.dot(x_ref[...], wlt_ref[...], preferred_element_type=jnp.float32,
                precision=_HIGH), 0.0)
    h_ref[...] = h
    g_ref[...] = jnp.dot(h, w1ht_ref[...], preferred_element_type=jnp.float32,
                         precision=_HIGH) + b1_ref[...]


def _edge_body(ea_ref, w1et_ref, out_ref):
    out_ref[...] = jnp.dot(ea_ref[...], w1et_ref[...],
                           preferred_element_type=jnp.float32, precision=_HIGH)


def _final_body(ns_ref, h_ref, w2nt_ref, w2ht_ref, b2_ref, o_ref):
    acc = jnp.dot(ns_ref[...], w2nt_ref[...],
                  preferred_element_type=jnp.float32, precision=_HIGH)
    acc = acc + jnp.dot(h_ref[...], w2ht_ref[...],
                        preferred_element_type=jnp.float32, precision=_HIGH)
    o_ref[...] = jnp.maximum(acc + b2_ref[...], 0.0)


def _sc_body(g2_hbm, ea3_hbm, src_hbm, dst_hbm, out_hbm,
             srcv0, ldst0, eabuf0, gbuf0,
             srcv1, ldst1, eabuf1, gbuf1,
             acc, sem0, sem1):
    c = lax.axis_index("c")
    s = lax.axis_index("s")
    base = c * OWN

    zero = jnp.zeros((L,), jnp.float32)
    zero_i = jnp.zeros((L,), jnp.int32)
    iota = jax.lax.iota(jnp.int32, L)

    bufs = ((srcv0, ldst0, eabuf0, gbuf0, sem0),
            (srcv1, ldst1, eabuf1, gbuf1, sem1))

    @pl.loop(0, SP_ROWS, unroll=8)
    def _zero(r):
        acc[r, pl.ds(0, L)] = zero

    def stage_issue(j, srcv, ldst, eabuf, gbuf, sem):
        pltpu.sync_copy(src_hbm.at[pl.ds(j, 1)], srcv)
        pltpu.sync_copy(dst_hbm.at[pl.ds(j, 1)], ldst)
        for a in range(C // L):
            sl = pl.ds(a * L, L)
            srcv[0, sl] = srcv[0, sl] * NSLICE + s
            d = ldst[0, sl] - base
            ok = (d >= 0) & (d < OWN)
            ldst[0, sl] = jnp.where(ok, d, DUMP)
        for q in range(GQ):
            pltpu.async_copy(g2_hbm.at[srcv.at[0, pl.ds(q * 128, 128)]],
                             gbuf.at[pl.ds(q * 128, 128)], sem)
        pltpu.async_copy(ea3_hbm.at[pl.ds(j * C, C), s], eabuf, sem)

    def drain(eabuf, gbuf, sem):
        pltpu.make_async_copy(g2_hbm.at[pl.ds(0, C)], gbuf, sem).wait()
        pltpu.make_async_copy(ea3_hbm.at[pl.ds(0, C), 0], eabuf, sem).wait()

    stage_issue(0, *bufs[0])

    @pl.loop(0, NCHUNK // 2)
    def _pair(g):
        j0 = g * 2
        for par in range(2):
            j = j0 + par
            srcv, ldst, eabuf, gbuf, sem = bufs[par]
            nsrcv, nldst, neabuf, ngbuf, nsem = bufs[1 - par]

            @pl.when(j + 1 < NCHUNK)
            def _prefetch():
                stage_issue(j + 1, nsrcv, nldst, neabuf, ngbuf, nsem)

            drain(eabuf, gbuf, sem)

            @pl.loop(0, C, unroll=8)
            def _edge(r):
                rowv = plsc.load_gather(
                    ldst, [zero_i, jnp.full((L,), r, jnp.int32)])
                m = jnp.maximum(gbuf[r, pl.ds(0, L)] + eabuf[r, pl.ds(0, L)],
                                0.0)
                plsc.addupdate_scatter(acc, [rowv, iota], m)

    pltpu.sync_copy(acc.at[pl.ds(0, OWN)], out_hbm.at[pl.ds(base, OWN), s])


def _sc_aggregate(g2, ea3, src2, dst2):
    mesh = plsc.VectorSubcoreMesh(core_axis_name="c", subcore_axis_name="s",
                                  num_cores=NC, num_subcores=NS)
    run = pl.kernel(
        _sc_body,
        out_type=jax.ShapeDtypeStruct((N_PAD, NSLICE, L), jnp.float32),
        mesh=mesh,
        scratch_types=[
            pltpu.VMEM((1, C), jnp.int32),
            pltpu.VMEM((1, C), jnp.int32),
            pltpu.VMEM((C, L), jnp.float32),
            pltpu.VMEM((C, L), jnp.float32),
            pltpu.VMEM((1, C), jnp.int32),
            pltpu.VMEM((1, C), jnp.int32),
            pltpu.VMEM((C, L), jnp.float32),
            pltpu.VMEM((C, L), jnp.float32),
            pltpu.VMEM((SP_ROWS, L), jnp.float32),
            pltpu.SemaphoreType.DMA,
            pltpu.SemaphoreType.DMA,
        ],
        compiler_params=pltpu.CompilerParams(needs_layout_passes=False,
                                             use_tc_tiling_on_sc=False),
    )
    return run(g2, ea3, src2, dst2)


def kernel(x, edge_index, edge_attr, W_lin, W1, b1, W2, b2):
    src2 = edge_index[0].astype(jnp.int32).reshape(NCHUNK, C)
    dst2 = edge_index[1].astype(jnp.int32).reshape(NCHUNK, C)

    wlt = W_lin.T
    w1ht = W1[:, :D].T
    w1et = W1[:, D:].T
    w2nt = W2[:, :D].T
    w2ht = W2[:, D:].T
    b1r = b1.reshape(1, D)
    b2r = b2.reshape(1, D)

    bn = 2000
    gn = N // bn
    full = pl.BlockSpec((None,) * 2, lambda i: (0, 0))

    h, g = pl.pallas_call(
        _front_body,
        grid=(gn,),
        in_specs=[
            pl.BlockSpec((bn, D), lambda i: (i, 0)),
            pl.BlockSpec((D, D), lambda i: (0, 0)),
            pl.BlockSpec((D, D), lambda i: (0, 0)),
            pl.BlockSpec((1, D), lambda i: (0, 0)),
        ],
        out_specs=[
            pl.BlockSpec((bn, D), lambda i: (i, 0)),
            pl.BlockSpec((bn, D), lambda i: (i, 0)),
        ],
        out_shape=[
            jax.ShapeDtypeStruct((N, D), jnp.float32),
            jax.ShapeDtypeStruct((N, D), jnp.float32),
        ],
    )(x, wlt, w1ht, b1r)

    be = 8000
    ea_proj = pl.pallas_call(
        _edge_body,
        grid=(E // be,),
        in_specs=[
            pl.BlockSpec((be, DE), lambda i: (i, 0)),
            pl.BlockSpec((DE, D), lambda i: (0, 0)),
        ],
        out_specs=pl.BlockSpec((be, D), lambda i: (i, 0)),
        out_shape=jax.ShapeDtypeStruct((E, D), jnp.float32),
    )(edge_attr, w1et)

    g2 = g.reshape(N * NSLICE, L)
    ea3 = ea_proj.reshape(E, NSLICE, L)
    out_blk = _sc_aggregate(g2, ea3, src2, dst2)
    ns = out_blk.reshape(N_PAD, D)[:N]

    out = pl.pallas_call(
        _final_body,
        grid=(gn,),
        in_specs=[
            pl.BlockSpec((bn, D), lambda i: (i, 0)),
            pl.BlockSpec((bn, D), lambda i: (i, 0)),
            pl.BlockSpec((D, D), lambda i: (0, 0)),
            pl.BlockSpec((D, D), lambda i: (0, 0)),
            pl.BlockSpec((1, D), lambda i: (0, 0)),
        ],
        out_specs=pl.BlockSpec((bn, D), lambda i: (i, 0)),
        out_shape=jax.ShapeDtypeStruct((N, D), jnp.float32),
    )(ns, h, w2nt, w2ht, b2r)

    return out


# speedup vs baseline: 2.1589x; 1.9353x over previous
import jax
import jax.numpy as jnp
from jax import lax
from jax.experimental import pallas as pl
from jax.experimental.pallas import tpu as pltpu
from jax.experimental.pallas import tpu_sc as plsc

N = 10000
E = 160000
D = 256
DE = 16

NC = 2
NS = 16
L = 16
NW = NC * NS

NSLICE = 16
OWN = 5120
N_PAD = NC * OWN
DUMP = OWN
SP_ROWS = OWN + 8

SEG = E // NW
CAP = 5120
PACK = 1 << 14
PAD_P = PACK * 0 + (PACK - 1)

C = 512
GQ = C // 128

_HIGH = lax.Precision.HIGHEST


def _front_body(x_ref, wlt_ref, w1ht_ref, b1_ref, h_ref, g_ref):
    h = jnp.maximum(
        jnp.dot(x_ref[...], wlt_ref[...], preferred_element_type=jnp.float32,
                precision=_HIGH), 0.0)
    h_ref[...] = h
    g_ref[...] = jnp.dot(h, w1ht_ref[...], preferred_element_type=jnp.float32,
                         precision=_HIGH) + b1_ref[...]


def _edge_body(ea_ref, w1et_ref, out_ref):
    out_ref[...] = jnp.dot(ea_ref[...], w1et_ref[...],
                           preferred_element_type=jnp.float32, precision=_HIGH)


def _final_body(ns_ref, h_ref, w2nt_ref, w2ht_ref, b2_ref, o_ref):
    acc = jnp.dot(ns_ref[...], w2nt_ref[...],
                  preferred_element_type=jnp.float32, precision=_HIGH)
    acc = acc + jnp.dot(h_ref[...], w2ht_ref[...],
                        preferred_element_type=jnp.float32, precision=_HIGH)
    o_ref[...] = jnp.maximum(acc + b2_ref[...], 0.0)


def _part_body(pk_hbm, out_hbm, ids_hbm, cnt_hbm,
               pbuf, outa, outb, idouta, idoutb, cbuf):
    c = lax.axis_index("c")
    s = lax.axis_index("s")
    w = c * NS + s

    iota = jax.lax.iota(jnp.int32, L)
    lane0 = iota == 0
    padv = jnp.full((L,), PAD_P, jnp.int32)
    zerov = jnp.zeros((L,), jnp.int32)
    zero_i = jnp.zeros((L,), jnp.int32)

    pltpu.sync_copy(pk_hbm.at[pl.ds(w, 1)], pbuf)

    @pl.loop(0, CAP // L, unroll=8)
    def _fill(a):
        sl = pl.ds(a * L, L)
        outa[0, sl] = padv
        outb[0, sl] = padv
        idouta[0, sl] = zerov
        idoutb[0, sl] = zerov

    wbase = jnp.full((L,), w * SEG, jnp.int32)

    def _scan(r, cnts):
        cnta, cntb = cnts
        p = plsc.load_gather(pbuf, [zero_i, jnp.full((L,), r, jnp.int32)])
        dv = jnp.bitwise_and(p, PACK - 1)
        isa = dv < OWN
        idv = wbase + r
        plsc.store_scatter(outa, [zero_i, cnta], p, mask=isa & lane0)
        plsc.store_scatter(idouta, [zero_i, cnta], idv, mask=isa & lane0)
        plsc.store_scatter(outb, [zero_i, cntb], p, mask=(~isa) & lane0)
        plsc.store_scatter(idoutb, [zero_i, cntb], idv, mask=(~isa) & lane0)
        return (cnta + isa.astype(jnp.int32), cntb + (~isa).astype(jnp.int32))

    cnta, cntb = lax.fori_loop(0, SEG, _scan, (zerov, zerov))

    cbuf[0, pl.ds(0, L)] = jnp.where(lane0, cnta, jnp.where(iota == 1, cntb, 0))

    pltpu.sync_copy(outa, out_hbm.at[0, pl.ds(w, 1)])
    pltpu.sync_copy(outb, out_hbm.at[1, pl.ds(w, 1)])
    pltpu.sync_copy(idouta, ids_hbm.at[0, pl.ds(w, 1)])
    pltpu.sync_copy(idoutb, ids_hbm.at[1, pl.ds(w, 1)])
    pltpu.sync_copy(cbuf, cnt_hbm.at[pl.ds(w, 1)])


def _sc_partition(pk):
    mesh = plsc.VectorSubcoreMesh(core_axis_name="c", subcore_axis_name="s",
                                  num_cores=NC, num_subcores=NS)
    run = pl.kernel(
        _part_body,
        out_type=[
            jax.ShapeDtypeStruct((NC, NW, CAP), jnp.int32),
            jax.ShapeDtypeStruct((NC, NW, CAP), jnp.int32),
            jax.ShapeDtypeStruct((NW, L), jnp.int32),
        ],
        mesh=mesh,
        scratch_types=[
            pltpu.VMEM((1, SEG), jnp.int32),
            pltpu.VMEM((1, CAP), jnp.int32),
            pltpu.VMEM((1, CAP), jnp.int32),
            pltpu.VMEM((1, CAP), jnp.int32),
            pltpu.VMEM((1, CAP), jnp.int32),
            pltpu.VMEM((1, L), jnp.int32),
        ],
        compiler_params=pltpu.CompilerParams(needs_layout_passes=False,
                                             use_tc_tiling_on_sc=False),
    )
    return run(pk)


def _sc_body(g2_hbm, ea2_hbm, pk_hbm, ids_hbm, cnt_hbm, out_hbm,
             pbuf0, idbuf0, ldst0, eabuf0, gbuf0,
             pbuf1, idbuf1, ldst1, eabuf1, gbuf1,
             cntbuf, acc, sem0, sem1):
    c = lax.axis_index("c")
    s = lax.axis_index("s")
    base = c * OWN

    zero = jnp.zeros((L,), jnp.float32)
    zero_i = jnp.zeros((L,), jnp.int32)
    iota = jax.lax.iota(jnp.int32, L)

    bufs = ((pbuf0, idbuf0, ldst0, eabuf0, gbuf0, sem0),
            (pbuf1, idbuf1, ldst1, eabuf1, gbuf1, sem1))

    pltpu.sync_copy(cnt_hbm, cntbuf)

    @pl.loop(0, SP_ROWS, unroll=8)
    def _zero(r):
        acc[r, pl.ds(0, L)] = zero

    def stage_issue(k, j, pbuf, idbuf, ldst, eabuf, gbuf, sem):
        pltpu.sync_copy(pk_hbm.at[c, pl.ds(k, 1), pl.ds(j * C, C)], pbuf)
        pltpu.sync_copy(ids_hbm.at[c, pl.ds(k, 1), pl.ds(j * C, C)], idbuf)

        @pl.loop(0, C // L, unroll=4)
        def _fix(a):
            sl = pl.ds(a * L, L)
            p = pbuf[0, sl]
            d = jnp.bitwise_and(p, PACK - 1) - base
            ok = (d >= 0) & (d < OWN)
            ldst[0, sl] = jnp.where(ok, d, DUMP)
            pbuf[0, sl] = lax.shift_right_logical(p, 14) * NSLICE + s
            idbuf[0, sl] = idbuf[0, sl] * NSLICE + s

        for q in range(GQ):
            qsl = pl.ds(q * 128, 128)
            pltpu.async_copy(g2_hbm.at[pbuf.at[0, qsl]], gbuf.at[qsl], sem)
            pltpu.async_copy(ea2_hbm.at[idbuf.at[0, qsl]], eabuf.at[qsl], sem)

    def drain(eabuf, gbuf, sem):
        pltpu.make_async_copy(g2_hbm.at[pl.ds(0, C)], gbuf, sem).wait()
        pltpu.make_async_copy(ea2_hbm.at[pl.ds(0, C)], eabuf, sem).wait()

    def accum(ldst, eabuf, gbuf):
        @pl.loop(0, C, unroll=8)
        def _edge(r):
            rowv = plsc.load_gather(
                ldst, [zero_i, jnp.full((L,), r, jnp.int32)])
            m = jnp.maximum(gbuf[r, pl.ds(0, L)] + eabuf[r, pl.ds(0, L)], 0.0)
            plsc.addupdate_scatter(acc, [rowv, iota], m)

    @pl.loop(0, NW)
    def _seg(k):
        cv = cntbuf[k, pl.ds(0, L)]
        cnt = jnp.max(jnp.where(iota == c, cv, 0))
        nch = (cnt + (C - 1)) // C

        @pl.when(nch > 0)
        def _prime():
            stage_issue(k, 0, *bufs[0])

        def _pair(gp, carry):
            j0 = gp * 2
            for par in range(2):
                j = j0 + par
                cur = bufs[par]
                nxt = bufs[1 - par]

                @pl.when(j < nch)
                def _do():
                    @pl.when(j + 1 < nch)
                    def _prefetch():
                        stage_issue(k, j + 1, *nxt)

                    drain(cur[3], cur[4], cur[5])
                    accum(cur[2], cur[3], cur[4])
            return carry

        lax.fori_loop(0, (nch + 1) // 2, _pair, 0)

    pltpu.sync_copy(acc.at[pl.ds(0, OWN)], out_hbm.at[pl.ds(base, OWN), s])


def _sc_aggregate(g2, ea2, pk, ids, cnts):
    mesh = plsc.VectorSubcoreMesh(core_axis_name="c", subcore_axis_name="s",
                                  num_cores=NC, num_subcores=NS)
    run = pl.kernel(
        _sc_body,
        out_type=jax.ShapeDtypeStruct((N_PAD, NSLICE, L), jnp.float32),
        mesh=mesh,
        scratch_types=[
            pltpu.VMEM((1, C), jnp.int32),
            pltpu.VMEM((1, C), jnp.int32),
            pltpu.VMEM((1, C), jnp.int32),
            pltpu.VMEM((C, L), jnp.float32),
            pltpu.VMEM((C, L), jnp.float32),
            pltpu.VMEM((1, C), jnp.int32),
            pltpu.VMEM((1, C), jnp.int32),
            pltpu.VMEM((1, C), jnp.int32),
            pltpu.VMEM((C, L), jnp.float32),
            pltpu.VMEM((C, L), jnp.float32),
            pltpu.VMEM((NW, L), jnp.int32),
            pltpu.VMEM((SP_ROWS, L), jnp.float32),
            pltpu.SemaphoreType.DMA,
            pltpu.SemaphoreType.DMA,
        ],
        compiler_params=pltpu.CompilerParams(needs_layout_passes=False,
                                             use_tc_tiling_on_sc=False),
    )
    return run(g2, ea2, pk, ids, cnts)


def kernel(x, edge_index, edge_attr, W_lin, W1, b1, W2, b2):
    src = edge_index[0].astype(jnp.int32)
    dst = edge_index[1].astype(jnp.int32)
    pk = (src * PACK + dst).reshape(NW, SEG)

    wlt = W_lin.T
    w1ht = W1[:, :D].T
    w1et = W1[:, D:].T
    w2nt = W2[:, :D].T
    w2ht = W2[:, D:].T
    b1r = b1.reshape(1, D)
    b2r = b2.reshape(1, D)

    bn = 2000
    gn = N // bn

    h, g = pl.pallas_call(
        _front_body,
        grid=(gn,),
        in_specs=[
            pl.BlockSpec((bn, D), lambda i: (i, 0)),
            pl.BlockSpec((D, D), lambda i: (0, 0)),
            pl.BlockSpec((D, D), lambda i: (0, 0)),
            pl.BlockSpec((1, D), lambda i: (0, 0)),
        ],
        out_specs=[
            pl.BlockSpec((bn, D), lambda i: (i, 0)),
            pl.BlockSpec((bn, D), lambda i: (i, 0)),
        ],
        out_shape=[
            jax.ShapeDtypeStruct((N, D), jnp.float32),
            jax.ShapeDtypeStruct((N, D), jnp.float32),
        ],
    )(x, wlt, w1ht, b1r)

    be = 8000
    ea_proj = pl.pallas_call(
        _edge_body,
        grid=(E // be,),
        in_specs=[
            pl.BlockSpec((be, DE), lambda i: (i, 0)),
            pl.BlockSpec((DE, D), lambda i: (0, 0)),
        ],
        out_specs=pl.BlockSpec((be, D), lambda i: (i, 0)),
        out_shape=jax.ShapeDtypeStruct((E, D), jnp.float32),
    )(edge_attr, w1et)

    pkc, ids, cnts = _sc_partition(pk)

    g2 = g.reshape(N * NSLICE, L)
    ea2 = ea_proj.reshape(E * NSLICE, L)
    out_blk = _sc_aggregate(g2, ea2, pkc, ids, cnts)
    ns = out_blk.reshape(N_PAD, D)[:N]

    out = pl.pallas_call(
        _final_body,
        grid=(gn,),
        in_specs=[
            pl.BlockSpec((bn, D), lambda i: (i, 0)),
            pl.BlockSpec((bn, D), lambda i: (i, 0)),
            pl.BlockSpec((D, D), lambda i: (0, 0)),
            pl.BlockSpec((D, D), lambda i: (0, 0)),
            pl.BlockSpec((1, D), lambda i: (0, 0)),
        ],
        out_specs=pl.BlockSpec((bn, D), lambda i: (i, 0)),
        out_shape=jax.ShapeDtypeStruct((N, D), jnp.float32),
    )(ns, h, w2nt, w2ht, b2r)

    return out
